# baseline (speedup 1.0000x reference)
import functools

import jax
import jax.numpy as jnp
from jax import lax
from jax.experimental import pallas as pl
from jax.experimental.pallas import tpu as pltpu
from jax.experimental.pallas import tpu_sc as plsc

_NUM_VALUES = 100000
_PAD_VALUES = 100352
_BATCH = 128
_HALF = _BATCH // 2
_WORDS = _PAD_VALUES // 4
_GROUP_WORDS = 256
_NGROUPS = _WORDS // _GROUP_WORDS
_GPB = 14
_NBLOCKS = _NGROUPS // _GPB
_NWORKERS = 32
_ROWS_PER_W = _HALF // _NWORKERS
_MUL = 0x01010101


def _byte_sums(w):
    return lax.shift_right_logical(w * _MUL, 24)


def _lane_extract(vec, lane):
    iota = lax.iota(jnp.int32, 16)
    zero = jnp.zeros((16,), vec.dtype)
    return jnp.sum(jnp.where(iota == lane, vec, zero))


_mesh = plsc.VectorSubcoreMesh(core_axis_name="c", subcore_axis_name="s")


@functools.partial(
    pl.kernel,
    out_type=(
        jax.ShapeDtypeStruct((_NWORKERS, 16), jnp.int32),
        jax.ShapeDtypeStruct((_NWORKERS, 16), jnp.float32),
    ),
    mesh=_mesh,
    compiler_params=pltpu.CompilerParams(needs_layout_passes=False),
    scratch_types=[
        pltpu.VMEM((_WORDS,), jnp.int32),
        pltpu.VMEM((_WORDS,), jnp.int32),
        pltpu.VMEM((_NGROUPS * 16,), jnp.int32),
        pltpu.SMEM((32,), jnp.int32),
        pltpu.VMEM((_HALF + 16,), jnp.float32),
        pltpu.VMEM((16,), jnp.float32),
        pltpu.VMEM((16,), jnp.int32),
        pltpu.SemaphoreType.DMA,
        pltpu.SemaphoreType.DMA,
    ],
)
def _sample_kernel(mask_hbm, u_hbm, cont_hbm, int_out, cont_out,
                   words_a, words_b, pbs, lvl1, u_all, cont_v, res,
                   sem_a, sem_b):
    wid = lax.axis_index("s") * 2 + lax.axis_index("c")
    bufs = (words_a, words_b)
    sems = (sem_a, sem_b)

    def row_copy(r):
        return pltpu.make_async_copy(mask_hbm.at[wid * _ROWS_PER_W + r],
                                     bufs[r % 2], sems[r % 2])

    for r in range(_ROWS_PER_W):
        row_copy(r).start()

    pltpu.sync_copy(u_hbm, u_all.at[pl.ds(0, _HALF)])

    pltpu.sync_copy(cont_hbm.at[wid], cont_v)
    x = cont_v[...]
    cont_v[...] = -1.0 + x * 2.0
    pltpu.sync_copy(cont_v, cont_out.at[wid])

    ubase = (wid // 8) * 16
    uvec = u_all[pl.ds(ubase, 16)]

    res_vec = jnp.zeros((16,), jnp.int32)
    for i in range(_ROWS_PER_W):
        row = wid * _ROWS_PER_W + i
        words = bufs[i % 2]
        row_copy(i).wait()

        def blk_body(j, tot):
            blk = jnp.zeros((16,), jnp.int32)
            base_g = j * _GPB
            for g2 in range(_GPB):
                base_w = (base_g + g2) * _GROUP_WORDS
                acc = words[pl.ds(base_w, 16)]
                for k in range(1, _GROUP_WORDS // 16):
                    acc = acc + words[pl.ds(base_w + k * 16, 16)]
                pb = _byte_sums(acc)
                pbs[pl.ds((base_g + g2) * 16, 16)] = pb
                blk = blk + pb
            s = jnp.sum(blk)
            lvl1[j] = s
            return tot + s

        nv = lax.fori_loop(0, _NBLOCKS, blk_body, jnp.int32(0))

        u = _lane_extract(uvec, row - ubase)
        p = u * nv.astype(jnp.float32)
        off0 = p.astype(jnp.int32)
        off0 = off0 - jnp.where(off0.astype(jnp.float32) > p,
                                jnp.int32(1), jnp.int32(0))
        off = jnp.minimum(off0, nv - jnp.int32(1))

        def cond1(c):
            j, acc = c
            return acc + lvl1[j] <= off

        def body1(c):
            j, acc = c
            return j + jnp.int32(1), acc + lvl1[j]

        jb, acc1 = lax.while_loop(cond1, body1, (jnp.int32(0), jnp.int32(0)))

        def cond2(c):
            g, acc = c
            return acc + jnp.sum(pbs[pl.ds(g * 16, 16)]) <= off

        def body2(c):
            g, acc = c
            return g + jnp.int32(1), acc + jnp.sum(pbs[pl.ds(g * 16, 16)])

        g, acc2 = lax.while_loop(cond2, body2, (jb * _GPB, acc1))

        def cond3(c):
            k, acc = c
            s = _byte_sums(words[pl.ds(g * _GROUP_WORDS + k * 16, 16)])
            return acc + jnp.sum(s) <= off

        def body3(c):
            k, acc = c
            s = _byte_sums(words[pl.ds(g * _GROUP_WORDS + k * 16, 16)])
            return k + jnp.int32(1), acc + jnp.sum(s)

        k, acc3 = lax.while_loop(cond3, body3, (jnp.int32(0), acc2))

        wbase = g * _GROUP_WORDS + k * 16
        wvec = words[pl.ds(wbase, 16)]
        s = _byte_sums(wvec)
        c = plsc.cumsum(s)
        sel = (acc3 + c) > off
        lane = jnp.max(plsc.all_reduce_ffs(sel))

        base_rank = _lane_extract(acc3 + c - s, lane)
        wv = _lane_extract(wvec, lane)
        rp = off - base_rank
        b0 = wv & jnp.int32(1)
        b1 = (wv >> 8) & jnp.int32(1)
        b2 = (wv >> 16) & jnp.int32(1)
        t1 = b0 + b1
        t2 = t1 + b2
        byte = jnp.where(rp < b0, jnp.int32(0),
                         jnp.where(rp < t1, jnp.int32(1),
                                   jnp.where(rp < t2, jnp.int32(2),
                                             jnp.int32(3))))
        col = (wbase + lane) * 4 + byte
        res_vec = jnp.where(lax.iota(jnp.int32, 16) == i, col, res_vec)

    res[...] = res_vec
    pltpu.sync_copy(res, int_out.at[wid])


def _pack_half(mask_half):
    pack_w = jnp.array([1, 1 << 8, 1 << 16, 1 << 24], jnp.int32)
    mp = jnp.pad(mask_half, ((0, 0), (0, _PAD_VALUES - _NUM_VALUES)))
    return jnp.einsum("bwk,k->bw", mp.reshape(_HALF, _WORDS, 4),
                      pack_w, preferred_element_type=jnp.int32)


def kernel(mask, uniform_int, uniform_cont):
    outs = []
    for h in range(2):
        rows = slice(h * _HALF, (h + 1) * _HALF)
        mwords = _pack_half(mask[rows])
        cont_in = uniform_cont[rows].reshape(_NWORKERS, 16)
        outs.append(_sample_kernel(mwords, uniform_int[rows], cont_in))
    int_action = jnp.concatenate(
        [o[0][:, :_ROWS_PER_W] for o in outs], axis=0).reshape(_BATCH)
    cont_action = jnp.concatenate(
        [o[1] for o in outs], axis=0).reshape(_BATCH, 8)
    return int_action, cont_action


# speedup vs baseline: 1.1817x; 1.1817x over previous
import functools

import jax
import jax.numpy as jnp
from jax import lax
from jax.experimental import pallas as pl
from jax.experimental.pallas import tpu as pltpu
from jax.experimental.pallas import tpu_sc as plsc

_NUM_VALUES = 100000
_PAD_VALUES = 100352
_BATCH = 128
_WORDS = _PAD_VALUES // 4
_GROUP_WORDS = 256
_NGROUPS = _WORDS // _GROUP_WORDS
_GPB = 14
_NBLOCKS = _NGROUPS // _GPB
_NWORKERS = 32
_ROWS_PER_W = _BATCH // _NWORKERS
_MUL = 0x01010101


def _byte_sums(w):
    return lax.shift_right_logical(w * _MUL, 24)


def _lane_extract(vec, lane):
    iota = lax.iota(jnp.int32, 16)
    zero = jnp.zeros((16,), vec.dtype)
    return jnp.sum(jnp.where(iota == lane, vec, zero))


_mesh = plsc.VectorSubcoreMesh(core_axis_name="c", subcore_axis_name="s")


@functools.partial(
    pl.kernel,
    out_type=(
        jax.ShapeDtypeStruct((_NWORKERS, 16), jnp.int32),
        jax.ShapeDtypeStruct((_NWORKERS, 32), jnp.float32),
    ),
    mesh=_mesh,
    compiler_params=pltpu.CompilerParams(needs_layout_passes=False),
    scratch_types=[
        pltpu.VMEM((_WORDS,), jnp.int32),
        pltpu.VMEM((_WORDS,), jnp.int32),
        pltpu.VMEM((_NGROUPS * 16,), jnp.int32),
        pltpu.SMEM((32,), jnp.int32),
        pltpu.VMEM((_BATCH + 16,), jnp.float32),
        pltpu.VMEM((32,), jnp.float32),
        pltpu.VMEM((16,), jnp.int32),
        pltpu.SemaphoreType.DMA,
        pltpu.SemaphoreType.DMA,
    ],
)
def _sample_kernel(mask_hbm, u_hbm, cont_hbm, int_out, cont_out,
                   words_a, words_b, pbs, lvl1, u_all, cont_v, res,
                   sem_a, sem_b):
    wid = lax.axis_index("s") * 2 + lax.axis_index("c")
    bufs = (words_a, words_b)
    sems = (sem_a, sem_b)

    def row_copy(r):
        return pltpu.make_async_copy(mask_hbm.at[wid * _ROWS_PER_W + r],
                                     bufs[r % 2], sems[r % 2])

    row_copy(0).start()
    row_copy(1).start()

    pltpu.sync_copy(u_hbm, u_all.at[pl.ds(0, _BATCH)])

    pltpu.sync_copy(cont_hbm.at[wid], cont_v)
    for h in range(2):
        x = cont_v[pl.ds(h * 16, 16)]
        cont_v[pl.ds(h * 16, 16)] = -1.0 + x * 2.0
    pltpu.sync_copy(cont_v, cont_out.at[wid])

    ubase = (wid // 4) * 16
    uvec = u_all[pl.ds(ubase, 16)]

    res_vec = jnp.zeros((16,), jnp.int32)
    for i in range(_ROWS_PER_W):
        row = wid * _ROWS_PER_W + i
        words = bufs[i % 2]
        row_copy(i).wait()

        def blk_body(j, tot):
            blk = jnp.zeros((16,), jnp.int32)
            base_g = j * _GPB
            for g2 in range(_GPB):
                base_w = (base_g + g2) * _GROUP_WORDS
                acc = words[pl.ds(base_w, 16)]
                for k in range(1, _GROUP_WORDS // 16):
                    acc = acc + words[pl.ds(base_w + k * 16, 16)]
                pb = _byte_sums(acc)
                pbs[pl.ds((base_g + g2) * 16, 16)] = pb
                blk = blk + pb
            s = jnp.sum(blk)
            lvl1[j] = s
            return tot + s

        nv = lax.fori_loop(0, _NBLOCKS, blk_body, jnp.int32(0))

        u = _lane_extract(uvec, row - ubase)
        p = u * nv.astype(jnp.float32)
        off0 = p.astype(jnp.int32)
        off0 = off0 - jnp.where(off0.astype(jnp.float32) > p,
                                jnp.int32(1), jnp.int32(0))
        off = jnp.minimum(off0, nv - jnp.int32(1))

        def cond1(c):
            j, acc = c
            return acc + lvl1[j] <= off

        def body1(c):
            j, acc = c
            return j + jnp.int32(1), acc + lvl1[j]

        jb, acc1 = lax.while_loop(cond1, body1, (jnp.int32(0), jnp.int32(0)))

        def cond2(c):
            g, acc = c
            return acc + jnp.sum(pbs[pl.ds(g * 16, 16)]) <= off

        def body2(c):
            g, acc = c
            return g + jnp.int32(1), acc + jnp.sum(pbs[pl.ds(g * 16, 16)])

        g, acc2 = lax.while_loop(cond2, body2, (jb * _GPB, acc1))

        def cond3(c):
            k, acc = c
            s = _byte_sums(words[pl.ds(g * _GROUP_WORDS + k * 16, 16)])
            return acc + jnp.sum(s) <= off

        def body3(c):
            k, acc = c
            s = _byte_sums(words[pl.ds(g * _GROUP_WORDS + k * 16, 16)])
            return k + jnp.int32(1), acc + jnp.sum(s)

        k, acc3 = lax.while_loop(cond3, body3, (jnp.int32(0), acc2))

        wbase = g * _GROUP_WORDS + k * 16
        wvec = words[pl.ds(wbase, 16)]
        s = _byte_sums(wvec)
        c = plsc.cumsum(s)
        sel = (acc3 + c) > off
        lane = jnp.max(plsc.all_reduce_ffs(sel))

        base_rank = _lane_extract(acc3 + c - s, lane)
        wv = _lane_extract(wvec, lane)
        rp = off - base_rank
        b0 = wv & jnp.int32(1)
        b1 = (wv >> 8) & jnp.int32(1)
        b2 = (wv >> 16) & jnp.int32(1)
        t1 = b0 + b1
        t2 = t1 + b2
        byte = jnp.where(rp < b0, jnp.int32(0),
                         jnp.where(rp < t1, jnp.int32(1),
                                   jnp.where(rp < t2, jnp.int32(2),
                                             jnp.int32(3))))
        col = (wbase + lane) * 4 + byte
        res_vec = jnp.where(lax.iota(jnp.int32, 16) == i, col, res_vec)
        if i + 2 < _ROWS_PER_W:
            row_copy(i + 2).start()

    res[...] = res_vec
    pltpu.sync_copy(res, int_out.at[wid])


def kernel(mask, uniform_int, uniform_cont):
    pack_w = jnp.array([1, 1 << 8, 1 << 16, 1 << 24], jnp.int32)
    mp = jnp.pad(mask, ((0, 0), (0, _PAD_VALUES - _NUM_VALUES)))
    mwords = jnp.einsum("bwk,k->bw", mp.reshape(_BATCH, _WORDS, 4),
                        pack_w, preferred_element_type=jnp.int32)
    cont_in = uniform_cont.reshape(_NWORKERS, 32)
    int_out, cont_out = _sample_kernel(mwords, uniform_int, cont_in)
    int_action = int_out[:, :_ROWS_PER_W].reshape(_BATCH)
    cont_action = cont_out.reshape(_BATCH, 8)
    return int_action, cont_action
